# Initial kernel scaffold; baseline (speedup 1.0000x reference)
import jax
import jax.numpy as jnp
from jax.experimental import pallas as pl


def kernel(node_feat, dynamicVariable, edge_index):
    raise NotImplementedError("write your pallas kernel here")


# speedup vs baseline: 5.5617x; 5.5617x over previous
import jax
import jax.numpy as jnp
from jax.experimental import pallas as pl

_N = 10000
_D = 128
_K = 16
_R = 400


def _knn_body(x_rows_ref, x_ref, sq_ref, idx_ref):
    i = pl.program_id(0)
    xr = x_rows_ref[...]
    xall = x_ref[...]
    prod = jax.lax.dot_general(
        xr, xall, (((1,), (1,)), ((), ())),
        preferred_element_type=jnp.float32)
    sq_r = jnp.sum(xr * xr, axis=1, keepdims=True)
    dist = sq_r + sq_ref[...] - 2.0 * prod
    col = jax.lax.broadcasted_iota(jnp.int32, (_R, _N), 1)
    row_g = jax.lax.broadcasted_iota(jnp.int32, (_R, _N), 0) + i * _R
    dist = jnp.where(col == row_g, dist + 1e9, dist)

    picks = []
    for _ in range(_K):
        idx = jnp.argmin(dist, axis=1).astype(jnp.int32)
        picks.append(idx[:, None])
        dist = jnp.where(col == idx[:, None], jnp.inf, dist)
    idx_ref[...] = jnp.concatenate(picks, axis=1)


def kernel(node_feat, dynamicVariable, edge_index):
    x = dynamicVariable
    sq = jnp.sum(x * x, axis=1)[None, :]
    idx = pl.pallas_call(
        _knn_body,
        grid=(_N // _R,),
        in_specs=[
            pl.BlockSpec((_R, _D), lambda i: (i, 0)),
            pl.BlockSpec((_N, _D), lambda i: (0, 0)),
            pl.BlockSpec((1, _N), lambda i: (0, 0)),
        ],
        out_specs=pl.BlockSpec((_R, _K), lambda i: (i, 0)),
        out_shape=jax.ShapeDtypeStruct((_N, _K), jnp.int32),
    )(x, x, sq)

    src = idx.reshape(-1)
    dst = jnp.repeat(jnp.arange(_N, dtype=src.dtype), _K)
    new_edges = jnp.stack([src, dst]).astype(jnp.int64)
    skip = jnp.allclose(node_feat, dynamicVariable)
    out_feat = jnp.where(skip, node_feat, dynamicVariable)
    out_edges = jnp.where(skip, edge_index, new_edges)
    return out_feat, out_edges
